# Initial kernel scaffold; baseline (speedup 1.0000x reference)
import jax
import jax.numpy as jnp
from jax.experimental import pallas as pl


def kernel(z_e, codebook):
    raise NotImplementedError("write your pallas kernel here")


# speedup vs baseline: 1.1774x; 1.1774x over previous
import functools

import jax
import jax.numpy as jnp
from jax import lax
from jax.experimental import pallas as pl
from jax.experimental.pallas import tpu as pltpu
from jax.experimental.pallas import tpu_sc as plsc

_NUM_CODES = 8192
_CODE_DIM = 64
_B = 18432

_BT = 512
_BC = 1024

_NC = 2
_NS = 16
_NW = _NC * _NS


def _argmin_body(z_ref, cb_ref, zn_ref, cbn_ref, idx_ref):
    z = z_ref[...]
    zn = zn_ref[...]
    halves = []
    chunks_per_half = (_NUM_CODES // 2) // _BC
    for h in range(2):
        best_d = jnp.full((_BT, 1), jnp.inf, jnp.float32)
        best_i = jnp.zeros((_BT, 1), jnp.int32)
        for cc in range(chunks_per_half):
            c = h * chunks_per_half + cc
            cb = cb_ref[pl.ds(c * _BC, _BC), :]
            dots = lax.dot_general(z, cb, (((1,), (1,)), ((), ())))
            d = (zn + cbn_ref[:, pl.ds(c * _BC, _BC)]) - 2.0 * dots
            m = jnp.min(d, axis=1, keepdims=True)
            ids = lax.broadcasted_iota(jnp.int32, (_BT, _BC), 1) + c * _BC
            li = jnp.min(jnp.where(d == m, ids, _NUM_CODES), axis=1,
                         keepdims=True)
            upd = m < best_d
            best_d = jnp.where(upd, m, best_d)
            best_i = jnp.where(upd, li, best_i)
        halves.append((best_d, best_i))
    (d0, i0), (d1, i1) = halves
    d0r = d0.astype(jnp.bfloat16).astype(jnp.float32)
    sel = d1 < d0r
    idx_ref[...] = jnp.where(sel, i1, i0)


def _tc_argmin(z_e, codebook, zn, cbn):
    return pl.pallas_call(
        _argmin_body,
        grid=(_B // _BT,),
        in_specs=[
            pl.BlockSpec((_BT, _CODE_DIM), lambda i: (i, 0)),
            pl.BlockSpec((_NUM_CODES, _CODE_DIM), lambda i: (0, 0)),
            pl.BlockSpec((_BT, 1), lambda i: (i, 0)),
            pl.BlockSpec((1, _NUM_CODES), lambda i: (0, 0)),
        ],
        out_specs=pl.BlockSpec((_BT, 1), lambda i: (i, 0)),
        out_shape=jax.ShapeDtypeStruct((_B, 1), jnp.int32),
    )(z_e, codebook, zn, cbn)


_B_PER_W = _B // _NW


def _sc_gather_body(table_hbm, idx_hbm, out_hbm, idx_v, rows_v, sem):
    wid = lax.axis_index("s") * _NC + lax.axis_index("c")
    base = wid * _B_PER_W
    pltpu.sync_copy(idx_hbm.at[pl.ds(base, _B_PER_W)], idx_v)
    pltpu.async_copy(table_hbm.at[idx_v], rows_v, sem).wait()
    pltpu.sync_copy(rows_v, out_hbm.at[pl.ds(base, _B_PER_W)])


def _sc_gather(codebook, indices):
    mesh = plsc.VectorSubcoreMesh(core_axis_name="c", subcore_axis_name="s")
    k = functools.partial(
        pl.kernel,
        mesh=mesh,
        out_type=jax.ShapeDtypeStruct((_B, _CODE_DIM), jnp.float32),
        scratch_types=[
            pltpu.VMEM((_B_PER_W,), jnp.int32),
            pltpu.VMEM((_B_PER_W, _CODE_DIM), jnp.float32),
            pltpu.SemaphoreType.DMA,
        ],
        compiler_params=pltpu.CompilerParams(use_tc_tiling_on_sc=False),
    )(_sc_gather_body)
    return k(codebook, indices)


def kernel(z_e, codebook):
    zn = jnp.sum(z_e * z_e, axis=1, keepdims=True)
    cbn = jnp.sum(codebook * codebook, axis=1, keepdims=True)
    idx2d = _tc_argmin(z_e, codebook, zn, cbn.T)
    indices = idx2d.reshape(_B)
    z_q = _sc_gather(codebook, indices)
    return (z_q, indices)


# speedup vs baseline: 1.2238x; 1.0394x over previous
import functools

import jax
import jax.numpy as jnp
from jax import lax
from jax.experimental import pallas as pl
from jax.experimental.pallas import tpu as pltpu
from jax.experimental.pallas import tpu_sc as plsc

_NUM_CODES = 8192
_CODE_DIM = 64
_B = 18432

_BT = 512
_BC = 1024

_NC = 2
_NS = 16
_NW = _NC * _NS


def _argmin_body(z_ref, cb_ref, zn_ref, cbn_ref, idx_ref):
    z2 = 2.0 * z_ref[...]
    zn = zn_ref[...]
    halves = []
    chunks_per_half = (_NUM_CODES // 2) // _BC
    for h in range(2):
        best_d = jnp.full((_BT, 1), jnp.inf, jnp.float32)
        best_i = jnp.zeros((_BT, 1), jnp.int32)
        for cc in range(chunks_per_half):
            c = h * chunks_per_half + cc
            cb = cb_ref[pl.ds(c * _BC, _BC), :]
            dots2 = lax.dot_general(z2, cb, (((1,), (1,)), ((), ())))
            d = (zn + cbn_ref[:, pl.ds(c * _BC, _BC)]) - dots2
            m = jnp.min(d, axis=1, keepdims=True)
            ids = lax.broadcasted_iota(jnp.int32, (_BT, _BC), 1)
            li = jnp.min(jnp.where(d == m, ids, _BC), axis=1,
                         keepdims=True) + c * _BC
            upd = m < best_d
            best_d = jnp.where(upd, m, best_d)
            best_i = jnp.where(upd, li, best_i)
        halves.append((best_d, best_i))
    (d0, i0), (d1, i1) = halves
    d0r = d0.astype(jnp.bfloat16).astype(jnp.float32)
    sel = d1 < d0r
    idx_ref[...] = jnp.where(sel, i1, i0)


def _tc_argmin(z_e, codebook, zn, cbn):
    return pl.pallas_call(
        _argmin_body,
        grid=(_B // _BT,),
        in_specs=[
            pl.BlockSpec((_BT, _CODE_DIM), lambda i: (i, 0)),
            pl.BlockSpec((_NUM_CODES, _CODE_DIM), lambda i: (0, 0)),
            pl.BlockSpec((_BT, 1), lambda i: (i, 0)),
            pl.BlockSpec((1, _NUM_CODES), lambda i: (0, 0)),
        ],
        out_specs=pl.BlockSpec((_BT, 1), lambda i: (i, 0)),
        out_shape=jax.ShapeDtypeStruct((_B, 1), jnp.int32),
    )(z_e, codebook, zn, cbn)


_B_PER_W = _B // _NW


def _sc_gather_body(table_hbm, idx_hbm, out_hbm, idx_v, rows_v, sem):
    wid = lax.axis_index("s") * _NC + lax.axis_index("c")
    base = wid * _B_PER_W
    pltpu.sync_copy(idx_hbm.at[pl.ds(base, _B_PER_W)], idx_v)
    pltpu.async_copy(table_hbm.at[idx_v], rows_v, sem).wait()
    pltpu.sync_copy(rows_v, out_hbm.at[pl.ds(base, _B_PER_W)])


def _sc_gather(codebook, indices):
    mesh = plsc.VectorSubcoreMesh(core_axis_name="c", subcore_axis_name="s")
    k = functools.partial(
        pl.kernel,
        mesh=mesh,
        out_type=jax.ShapeDtypeStruct((_B, _CODE_DIM), jnp.float32),
        scratch_types=[
            pltpu.VMEM((_B_PER_W,), jnp.int32),
            pltpu.VMEM((_B_PER_W, _CODE_DIM), jnp.float32),
            pltpu.SemaphoreType.DMA,
        ],
        compiler_params=pltpu.CompilerParams(use_tc_tiling_on_sc=False),
    )(_sc_gather_body)
    return k(codebook, indices)


def kernel(z_e, codebook):
    zn = jnp.sum(z_e * z_e, axis=1, keepdims=True)
    cbn = jnp.sum(codebook * codebook, axis=1, keepdims=True)
    idx2d = _tc_argmin(z_e, codebook, zn, cbn.T)
    indices = idx2d.reshape(_B)
    z_q = _sc_gather(codebook, indices)
    return (z_q, indices)


# speedup vs baseline: 1.2691x; 1.0370x over previous
import functools

import jax
import jax.numpy as jnp
from jax import lax
from jax.experimental import pallas as pl
from jax.experimental.pallas import tpu as pltpu
from jax.experimental.pallas import tpu_sc as plsc

_NUM_CODES = 8192
_CODE_DIM = 64
_B = 18432

_BT = 1024
_BC = 1024

_NC = 2
_NS = 16
_NW = _NC * _NS


def _argmin_body(z_ref, cb_ref, zn_ref, cbn_ref, idx_ref):
    z2 = 2.0 * z_ref[...]
    zn = zn_ref[...]
    halves = []
    chunks_per_half = (_NUM_CODES // 2) // _BC
    for h in range(2):
        best_d = jnp.full((_BT, 1), jnp.inf, jnp.float32)
        best_i = jnp.zeros((_BT, 1), jnp.int32)
        for cc in range(chunks_per_half):
            c = h * chunks_per_half + cc
            cb = cb_ref[pl.ds(c * _BC, _BC), :]
            dots2 = lax.dot_general(z2, cb, (((1,), (1,)), ((), ())))
            d = (zn + cbn_ref[:, pl.ds(c * _BC, _BC)]) - dots2
            m = jnp.min(d, axis=1, keepdims=True)
            ids = lax.broadcasted_iota(jnp.int32, (_BT, _BC), 1)
            li = jnp.min(jnp.where(d == m, ids, _BC), axis=1,
                         keepdims=True) + c * _BC
            upd = m < best_d
            best_d = jnp.where(upd, m, best_d)
            best_i = jnp.where(upd, li, best_i)
        halves.append((best_d, best_i))
    (d0, i0), (d1, i1) = halves
    d0r = d0.astype(jnp.bfloat16).astype(jnp.float32)
    sel = d1 < d0r
    idx_ref[...] = jnp.where(sel, i1, i0)


def _tc_argmin(z_e, codebook, zn, cbn):
    return pl.pallas_call(
        _argmin_body,
        grid=(_B // _BT,),
        in_specs=[
            pl.BlockSpec((_BT, _CODE_DIM), lambda i: (i, 0)),
            pl.BlockSpec((_NUM_CODES, _CODE_DIM), lambda i: (0, 0)),
            pl.BlockSpec((_BT, 1), lambda i: (i, 0)),
            pl.BlockSpec((1, _NUM_CODES), lambda i: (0, 0)),
        ],
        out_specs=pl.BlockSpec((_BT, 1), lambda i: (i, 0)),
        out_shape=jax.ShapeDtypeStruct((_B, 1), jnp.int32),
    )(z_e, codebook, zn, cbn)


_B_PER_W = _B // _NW


def _sc_gather_body(table_hbm, idx_hbm, out_hbm, idx_v, rows_v, sem):
    wid = lax.axis_index("s") * _NC + lax.axis_index("c")
    base = wid * _B_PER_W
    pltpu.sync_copy(idx_hbm.at[pl.ds(base, _B_PER_W)], idx_v)
    pltpu.async_copy(table_hbm.at[idx_v], rows_v, sem).wait()
    pltpu.sync_copy(rows_v, out_hbm.at[pl.ds(base, _B_PER_W)])


def _sc_gather(codebook, indices):
    mesh = plsc.VectorSubcoreMesh(core_axis_name="c", subcore_axis_name="s")
    k = functools.partial(
        pl.kernel,
        mesh=mesh,
        out_type=jax.ShapeDtypeStruct((_B, _CODE_DIM), jnp.float32),
        scratch_types=[
            pltpu.VMEM((_B_PER_W,), jnp.int32),
            pltpu.VMEM((_B_PER_W, _CODE_DIM), jnp.float32),
            pltpu.SemaphoreType.DMA,
        ],
        compiler_params=pltpu.CompilerParams(use_tc_tiling_on_sc=False),
    )(_sc_gather_body)
    return k(codebook, indices)


def kernel(z_e, codebook):
    zn = jnp.sum(z_e * z_e, axis=1, keepdims=True)
    cbn = jnp.sum(codebook * codebook, axis=1, keepdims=True)
    idx2d = _tc_argmin(z_e, codebook, zn, cbn.T)
    indices = idx2d.reshape(_B)
    z_q = _sc_gather(codebook, indices)
    return (z_q, indices)
